# baseline (device time: 8578 ns/iter reference)
import jax
import jax.numpy as jnp
from jax import lax
from jax.experimental import pallas as pl
from jax.experimental.pallas import tpu as pltpu


def kernel(x):
    m, n = x.shape
    half = n // 2

    def body(x_ref, out_ref, comm_ref, send_sem, recv_sem):
        my_x = lax.axis_index("x")
        my_y = lax.axis_index("y")
        my_z = lax.axis_index("z")
        other = 1 - my_z

        barrier_sem = pltpu.get_barrier_semaphore()
        pl.semaphore_signal(
            barrier_sem, inc=1,
            device_id=(my_x, my_y, other),
            device_id_type=pl.DeviceIdType.MESH,
        )
        comm_ref[:, :] = x_ref[:, pl.ds(other * half, half)]
        pl.semaphore_wait(barrier_sem, 1)

        rdma = pltpu.make_async_remote_copy(
            src_ref=comm_ref,
            dst_ref=out_ref.at[pl.ds(my_z * m, m), :],
            send_sem=send_sem,
            recv_sem=recv_sem,
            device_id=(my_x, my_y, other),
            device_id_type=pl.DeviceIdType.MESH,
        )
        rdma.start()

        out_ref[pl.ds(my_z * m, m), :] = x_ref[:, pl.ds(my_z * half, half)]

        rdma.wait()

    out_shape = jax.ShapeDtypeStruct((2 * m, half), x.dtype)
    return pl.pallas_call(
        body,
        out_shape=out_shape,
        in_specs=[pl.BlockSpec(memory_space=pltpu.VMEM)],
        out_specs=pl.BlockSpec(memory_space=pltpu.VMEM),
        scratch_shapes=[
            pltpu.VMEM((m, half), x.dtype),
            pltpu.SemaphoreType.DMA,
            pltpu.SemaphoreType.DMA,
        ],
        compiler_params=pltpu.CompilerParams(collective_id=0),
    )(x)


# device time: 8577 ns/iter; 1.0001x vs baseline; 1.0001x over previous
import jax
import jax.numpy as jnp
from jax import lax
from jax.experimental import pallas as pl
from jax.experimental.pallas import tpu as pltpu


def kernel(x):
    m, n = x.shape
    half = n // 2

    def body(x_ref, out_ref, comm_ref, send_sem, recv_sem):
        my_x = lax.axis_index("x")
        my_y = lax.axis_index("y")
        my_z = lax.axis_index("z")
        other = 1 - my_z

        barrier_sem = pltpu.get_barrier_semaphore()
        pl.semaphore_signal(
            barrier_sem, inc=1,
            device_id=(my_x, my_y, other),
            device_id_type=pl.DeviceIdType.MESH,
        )
        comm_ref[:, :] = x_ref[:, pl.ds(other * half, half)]
        pl.semaphore_wait(barrier_sem, 1)

        mh = m // 2
        rdmas = []
        for s in range(2):
            rdma = pltpu.make_async_remote_copy(
                src_ref=comm_ref.at[pl.ds(s * mh, mh), :],
                dst_ref=out_ref.at[pl.ds(my_z * m + s * mh, mh), :],
                send_sem=send_sem.at[s],
                recv_sem=recv_sem.at[s],
                device_id=(my_x, my_y, other),
                device_id_type=pl.DeviceIdType.MESH,
            )
            rdma.start()
            rdmas.append(rdma)

        out_ref[pl.ds(my_z * m, m), :] = x_ref[:, pl.ds(my_z * half, half)]

        for rdma in rdmas:
            rdma.wait()

    out_shape = jax.ShapeDtypeStruct((2 * m, half), x.dtype)
    return pl.pallas_call(
        body,
        out_shape=out_shape,
        in_specs=[pl.BlockSpec(memory_space=pltpu.VMEM)],
        out_specs=pl.BlockSpec(memory_space=pltpu.VMEM),
        scratch_shapes=[
            pltpu.VMEM((m, half), x.dtype),
            pltpu.SemaphoreType.DMA((2,)),
            pltpu.SemaphoreType.DMA((2,)),
        ],
        compiler_params=pltpu.CompilerParams(collective_id=0),
    )(x)
